# Initial kernel scaffold; baseline (speedup 1.0000x reference)
import functools

import jax
import jax.numpy as jnp
from jax import lax
from jax.experimental import pallas as pl
from jax.experimental.pallas import tpu as pltpu
from jax.experimental.pallas import tpu_sc as plsc

N = 10000
NPAD = 10240
PT = NPAD // 16
E = 320000
CHUNK = 128
NW = 32
CHPW = 79
EPAD = NW * CHPW * CHUNK
G = 64
IN_CH = 128
HID = 512
OUT_CH = 64


def _make_sc_prop(C):
  mesh = plsc.VectorSubcoreMesh(core_axis_name="c", subcore_axis_name="s")

  @functools.partial(
      pl.kernel,
      out_type=(jax.ShapeDtypeStruct((NPAD, C), jnp.float32),
                jax.ShapeDtypeStruct((NPAD, C), jnp.float32)),
      mesh=mesh,
      scratch_types=[
          pltpu.VMEM_SHARED((NPAD, C), jnp.float32),
          pltpu.VMEM((CHUNK,), jnp.int32),
          pltpu.VMEM((CHUNK,), jnp.int32),
          pltpu.VMEM((CHUNK, C), jnp.float32),
      ],
  )
  def prop(u_hbm, src_hbm, dst_hbm, zeros_hbm, out0, out1,
           acc_sh, sidx, didx, rows):
    cid = lax.axis_index("c")
    sid = lax.axis_index("s")
    pltpu.sync_copy(zeros_hbm, acc_sh.at[pl.ds(sid * PT, PT)])
    plsc.subcore_barrier()
    wid = cid * 16 + sid

    def body(k, carry):
      base = (wid * CHPW + k) * CHUNK
      pltpu.sync_copy(src_hbm.at[pl.ds(base, CHUNK)], sidx)
      pltpu.sync_copy(dst_hbm.at[pl.ds(base, CHUNK)], didx)
      pltpu.sync_copy(u_hbm.at[sidx], rows)
      pltpu.sync_copy(rows, acc_sh.at[didx], add=True)
      return carry

    lax.fori_loop(0, CHPW, body, 0)
    plsc.subcore_barrier()

    @pl.when(cid == 0)
    def _():
      pltpu.sync_copy(acc_sh.at[pl.ds(sid * PT, PT)],
                      out0.at[pl.ds(sid * PT, PT)])

    @pl.when(cid == 1)
    def _():
      pltpu.sync_copy(acc_sh.at[pl.ds(sid * PT, PT)],
                      out1.at[pl.ds(sid * PT, PT)])

  return prop


def _make_sc_deg():
  mesh = plsc.VectorSubcoreMesh(core_axis_name="c", subcore_axis_name="s")
  C = 16

  @functools.partial(
      pl.kernel,
      out_type=(jax.ShapeDtypeStruct((NPAD, C), jnp.float32),
                jax.ShapeDtypeStruct((NPAD, C), jnp.float32)),
      mesh=mesh,
      scratch_types=[
          pltpu.VMEM_SHARED((NPAD, C), jnp.float32),
          pltpu.VMEM((CHUNK,), jnp.int32),
          pltpu.VMEM((CHUNK, C), jnp.float32),
      ],
  )
  def deg(dst_hbm, ones_hbm, zeros_hbm, out0, out1, acc_sh, didx, ones_v):
    cid = lax.axis_index("c")
    sid = lax.axis_index("s")
    pltpu.sync_copy(zeros_hbm, acc_sh.at[pl.ds(sid * PT, PT)])
    pltpu.sync_copy(ones_hbm, ones_v)
    plsc.subcore_barrier()
    wid = cid * 16 + sid

    def body(k, carry):
      base = (wid * CHPW + k) * CHUNK
      pltpu.sync_copy(dst_hbm.at[pl.ds(base, CHUNK)], didx)
      pltpu.sync_copy(ones_v, acc_sh.at[didx], add=True)
      return carry

    lax.fori_loop(0, CHPW, body, 0)
    plsc.subcore_barrier()

    @pl.when(cid == 0)
    def _():
      pltpu.sync_copy(acc_sh.at[pl.ds(sid * PT, PT)],
                      out0.at[pl.ds(sid * PT, PT)])

    @pl.when(cid == 1)
    def _():
      pltpu.sync_copy(acc_sh.at[pl.ds(sid * PT, PT)],
                      out1.at[pl.ds(sid * PT, PT)])

  return deg


_RB = 1024


def _tc_prescale_body(x_ref, d0_ref, d1_ref, u_ref, dinv_ref):
  dinv = lax.rsqrt(1.0 + d0_ref[:, :1] + d1_ref[:, :1])
  u_ref[...] = x_ref[...] * dinv
  dinv_ref[...] = jnp.broadcast_to(dinv, dinv_ref.shape)


def _tc_layer1_body(u1_ref, a0_ref, a1_ref, dinv_ref, w1_ref, b1_ref, w2_ref,
                    u2_ref):
  i = pl.program_id(0)
  dinv = dinv_ref[:, :1]
  p1 = dinv * (u1_ref[...] + a0_ref[...] + a1_ref[...])
  h1 = jnp.maximum(
      jnp.dot(p1, w1_ref[...], preferred_element_type=jnp.float32)
      + b1_ref[...], 0.0)
  t = jnp.dot(h1, w2_ref[...], preferred_element_type=jnp.float32)
  row = i * _RB + lax.broadcasted_iota(jnp.int32, (_RB, 1), 0)
  u2_ref[...] = jnp.where(row < N, dinv * t, 0.0)


def _tc_pool_body(u2_ref, c0_ref, c1_ref, dinv_ref, b2_ref, batch_ref,
                  o_ref, cnt_ref):
  i = pl.program_id(0)
  nsteps = pl.num_programs(0)
  dinv = dinv_ref[:, :1]
  p2 = dinv * (u2_ref[...] + c0_ref[...] + c1_ref[...])
  h2 = jnp.maximum(p2 + b2_ref[...], 0.0)
  row = i * _RB + lax.broadcasted_iota(jnp.int32, (_RB, 1), 0)
  h2 = jnp.where(row < N, h2, 0.0)
  m = (batch_ref[...] ==
       lax.broadcasted_iota(jnp.int32, (1, G), 1)).astype(jnp.float32)
  part = lax.dot_general(m, h2, (((0,), (0,)), ((), ())),
                         preferred_element_type=jnp.float32)
  pcnt = lax.dot_general(m, jnp.ones((_RB, 1), jnp.float32),
                         (((0,), (0,)), ((), ())),
                         preferred_element_type=jnp.float32)

  @pl.when(i == 0)
  def _():
    o_ref[...] = jnp.zeros_like(o_ref)
    cnt_ref[...] = jnp.zeros_like(cnt_ref)

  o_ref[...] += part
  cnt_ref[:, :1] += pcnt

  @pl.when(i == nsteps - 1)
  def _():
    o_ref[...] = o_ref[...] / jnp.maximum(cnt_ref[:, :1], 1.0)


def kernel(x, edge_index, batch, W1, b1, W2, b2):
  f32 = jnp.float32
  pad_e = EPAD - E
  src_p = jnp.concatenate(
      [edge_index[0], jnp.full((pad_e,), N, jnp.int32)])
  dst_p = jnp.concatenate(
      [edge_index[1], jnp.full((pad_e,), N, jnp.int32)])
  x_p = jnp.pad(x, ((0, NPAD - N), (0, 0)))
  batch_p = jnp.concatenate(
      [batch, jnp.full((NPAD - N,), G, jnp.int32)]).reshape(NPAD, 1)
  b1r = b1.reshape(1, HID)
  b2r = b2.reshape(1, OUT_CH)
  zeros16 = jnp.zeros((PT, 16), f32)
  zeros128 = jnp.zeros((PT, IN_CH), f32)
  zeros64 = jnp.zeros((PT, OUT_CH), f32)
  ones16 = jnp.ones((CHUNK, 16), f32)

  d0, d1 = _make_sc_deg()(dst_p, ones16, zeros16)

  grid = NPAD // _RB
  u1, dinv16 = pl.pallas_call(
      _tc_prescale_body,
      grid=(grid,),
      in_specs=[
          pl.BlockSpec((_RB, IN_CH), lambda i: (i, 0)),
          pl.BlockSpec((_RB, 16), lambda i: (i, 0)),
          pl.BlockSpec((_RB, 16), lambda i: (i, 0)),
      ],
      out_specs=(pl.BlockSpec((_RB, IN_CH), lambda i: (i, 0)),
                 pl.BlockSpec((_RB, 16), lambda i: (i, 0))),
      out_shape=(jax.ShapeDtypeStruct((NPAD, IN_CH), f32),
                 jax.ShapeDtypeStruct((NPAD, 16), f32)),
  )(x_p, d0, d1)

  a0, a1 = _make_sc_prop(IN_CH)(u1, src_p, dst_p, zeros128)

  u2 = pl.pallas_call(
      _tc_layer1_body,
      grid=(grid,),
      in_specs=[
          pl.BlockSpec((_RB, IN_CH), lambda i: (i, 0)),
          pl.BlockSpec((_RB, IN_CH), lambda i: (i, 0)),
          pl.BlockSpec((_RB, IN_CH), lambda i: (i, 0)),
          pl.BlockSpec((_RB, 16), lambda i: (i, 0)),
          pl.BlockSpec((IN_CH, HID), lambda i: (0, 0)),
          pl.BlockSpec((1, HID), lambda i: (0, 0)),
          pl.BlockSpec((HID, OUT_CH), lambda i: (0, 0)),
      ],
      out_specs=pl.BlockSpec((_RB, OUT_CH), lambda i: (i, 0)),
      out_shape=jax.ShapeDtypeStruct((NPAD, OUT_CH), f32),
  )(u1, a0, a1, dinv16, W1, b1r, W2)

  c0, c1 = _make_sc_prop(OUT_CH)(u2, src_p, dst_p, zeros64)

  out = pl.pallas_call(
      _tc_pool_body,
      grid=(grid,),
      in_specs=[
          pl.BlockSpec((_RB, OUT_CH), lambda i: (i, 0)),
          pl.BlockSpec((_RB, OUT_CH), lambda i: (i, 0)),
          pl.BlockSpec((_RB, OUT_CH), lambda i: (i, 0)),
          pl.BlockSpec((_RB, 16), lambda i: (i, 0)),
          pl.BlockSpec((1, OUT_CH), lambda i: (0, 0)),
          pl.BlockSpec((_RB, 1), lambda i: (i, 0)),
      ],
      out_specs=pl.BlockSpec((G, OUT_CH), lambda i: (0, 0)),
      out_shape=jax.ShapeDtypeStruct((G, OUT_CH), f32),
      scratch_shapes=[pltpu.VMEM((G, 128), f32)],
  )(u2, c0, c1, dinv16, b2r, batch_p)
  return out


# speedup vs baseline: 16.7346x; 16.7346x over previous
import functools

import jax
import jax.numpy as jnp
from jax import lax
from jax.experimental import pallas as pl
from jax.experimental.pallas import tpu as pltpu
from jax.experimental.pallas import tpu_sc as plsc

N = 10000
NPAD = 10240
PT = NPAD // 16
E = 320000
CHUNK = 128
NW = 32
CHPW = 79
EPAD = NW * CHPW * CHUNK
G = 64
IN_CH = 128
HID = 512
OUT_CH = 64


def _make_sc_prop(C):
  mesh = plsc.VectorSubcoreMesh(core_axis_name="c", subcore_axis_name="s")

  @functools.partial(
      pl.kernel,
      out_type=jax.ShapeDtypeStruct((2, NPAD, C), jnp.float32),
      mesh=mesh,
      scratch_types=[
          pltpu.VMEM_SHARED((NPAD, C), jnp.float32),
          pltpu.VMEM((CHUNK,), jnp.int32),
          pltpu.VMEM((CHUNK,), jnp.int32),
          pltpu.VMEM((CHUNK, C), jnp.float32),
      ],
      compiler_params=pltpu.CompilerParams(use_tc_tiling_on_sc=(C == 128)),
  )
  def prop(u_hbm, src_hbm, dst_hbm, zeros_hbm, out, acc_sh, sidx, didx, rows):
    cid = lax.axis_index("c")
    sid = lax.axis_index("s")
    pltpu.sync_copy(zeros_hbm, acc_sh.at[pl.ds(sid * PT, PT)])
    plsc.subcore_barrier()
    wid = cid * 16 + sid

    def body(k, carry):
      base = (wid * CHPW + k) * CHUNK
      pltpu.sync_copy(src_hbm.at[pl.ds(base, CHUNK)], sidx)
      pltpu.sync_copy(dst_hbm.at[pl.ds(base, CHUNK)], didx)
      pltpu.sync_copy(u_hbm.at[sidx], rows)
      pltpu.sync_copy(rows, acc_sh.at[didx], add=True)
      return carry

    lax.fori_loop(0, CHPW, body, 0)
    plsc.subcore_barrier()
    pltpu.sync_copy(acc_sh.at[pl.ds(sid * PT, PT)],
                    out.at[cid, pl.ds(sid * PT, PT)])

  return prop


def _make_sc_deg():
  mesh = plsc.VectorSubcoreMesh(core_axis_name="c", subcore_axis_name="s")
  C = 16

  @functools.partial(
      pl.kernel,
      out_type=jax.ShapeDtypeStruct((2, NPAD, C), jnp.float32),
      mesh=mesh,
      scratch_types=[
          pltpu.VMEM_SHARED((NPAD, C), jnp.float32),
          pltpu.VMEM((CHUNK,), jnp.int32),
          pltpu.VMEM((CHUNK, C), jnp.float32),
      ],
      compiler_params=pltpu.CompilerParams(use_tc_tiling_on_sc=False),
  )
  def deg(dst_hbm, ones_hbm, zeros_hbm, out, acc_sh, didx, ones_v):
    cid = lax.axis_index("c")
    sid = lax.axis_index("s")
    pltpu.sync_copy(zeros_hbm, acc_sh.at[pl.ds(sid * PT, PT)])
    pltpu.sync_copy(ones_hbm, ones_v)
    plsc.subcore_barrier()
    wid = cid * 16 + sid

    def body(k, carry):
      base = (wid * CHPW + k) * CHUNK
      pltpu.sync_copy(dst_hbm.at[pl.ds(base, CHUNK)], didx)
      pltpu.sync_copy(ones_v, acc_sh.at[didx], add=True)
      return carry

    lax.fori_loop(0, CHPW, body, 0)
    plsc.subcore_barrier()
    pltpu.sync_copy(acc_sh.at[pl.ds(sid * PT, PT)],
                    out.at[cid, pl.ds(sid * PT, PT)])

  return deg


_RB = 1024


def _tc_prescale_body(x_ref, d_ref, u_ref, dinv_ref):
  dinv = lax.rsqrt(1.0 + d_ref[0, :, :1] + d_ref[1, :, :1])
  u_ref[...] = x_ref[...] * dinv
  dinv_ref[...] = jnp.broadcast_to(dinv, dinv_ref.shape)


def _tc_layer1_body(u1_ref, a_ref, dinv_ref, w1_ref, b1_ref, w2_ref, u2_ref):
  i = pl.program_id(0)
  dinv = dinv_ref[:, :1]
  p1 = dinv * (u1_ref[...] + a_ref[0] + a_ref[1])
  h1 = jnp.maximum(
      jnp.dot(p1, w1_ref[...], preferred_element_type=jnp.float32)
      + b1_ref[...], 0.0)
  t = jnp.dot(h1, w2_ref[...], preferred_element_type=jnp.float32)
  row = i * _RB + lax.broadcasted_iota(jnp.int32, (_RB, 1), 0)
  u2_ref[...] = jnp.where(row < N, dinv * t, 0.0)


def _tc_pool_body(u2_ref, c_ref, dinv_ref, b2_ref, batch_ref, o_ref, cnt_ref):
  i = pl.program_id(0)
  nsteps = pl.num_programs(0)
  dinv = dinv_ref[:, :1]
  p2 = dinv * (u2_ref[...] + c_ref[0] + c_ref[1])
  h2 = jnp.maximum(p2 + b2_ref[...], 0.0)
  row = i * _RB + lax.broadcasted_iota(jnp.int32, (_RB, 1), 0)
  h2 = jnp.where(row < N, h2, 0.0)
  m = (batch_ref[...] ==
       lax.broadcasted_iota(jnp.int32, (1, G), 1)).astype(jnp.float32)
  part = lax.dot_general(m, h2, (((0,), (0,)), ((), ())),
                         preferred_element_type=jnp.float32)
  pcnt = lax.dot_general(m, jnp.ones((_RB, 1), jnp.float32),
                         (((0,), (0,)), ((), ())),
                         preferred_element_type=jnp.float32)

  @pl.when(i == 0)
  def _():
    o_ref[...] = jnp.zeros_like(o_ref)
    cnt_ref[...] = jnp.zeros_like(cnt_ref)

  o_ref[...] += part
  cnt_ref[:, :1] += pcnt

  @pl.when(i == nsteps - 1)
  def _():
    o_ref[...] = o_ref[...] / jnp.maximum(cnt_ref[:, :1], 1.0)


def kernel(x, edge_index, batch, W1, b1, W2, b2):
  f32 = jnp.float32
  pad_e = EPAD - E
  src_p = jnp.concatenate(
      [edge_index[0], jnp.full((pad_e,), N, jnp.int32)])
  dst_p = jnp.concatenate(
      [edge_index[1], jnp.full((pad_e,), N, jnp.int32)])
  x_p = jnp.pad(x, ((0, NPAD - N), (0, 0)))
  batch_p = jnp.concatenate(
      [batch, jnp.full((NPAD - N,), G, jnp.int32)]).reshape(NPAD, 1)
  b1r = b1.reshape(1, HID)
  b2r = b2.reshape(1, OUT_CH)
  zeros16 = jnp.zeros((PT, 16), f32)
  zeros128 = jnp.zeros((PT, IN_CH), f32)
  zeros64 = jnp.zeros((PT, OUT_CH), f32)
  ones16 = jnp.ones((CHUNK, 16), f32)

  d = _make_sc_deg()(dst_p, ones16, zeros16)

  grid = NPAD // _RB
  u1, dinv16 = pl.pallas_call(
      _tc_prescale_body,
      grid=(grid,),
      in_specs=[
          pl.BlockSpec((_RB, IN_CH), lambda i: (i, 0)),
          pl.BlockSpec((2, _RB, 16), lambda i: (0, i, 0)),
      ],
      out_specs=(pl.BlockSpec((_RB, IN_CH), lambda i: (i, 0)),
                 pl.BlockSpec((_RB, 16), lambda i: (i, 0))),
      out_shape=(jax.ShapeDtypeStruct((NPAD, IN_CH), f32),
                 jax.ShapeDtypeStruct((NPAD, 16), f32)),
  )(x_p, d)

  a = _make_sc_prop(IN_CH)(u1, src_p, dst_p, zeros128)

  u2 = pl.pallas_call(
      _tc_layer1_body,
      grid=(grid,),
      in_specs=[
          pl.BlockSpec((_RB, IN_CH), lambda i: (i, 0)),
          pl.BlockSpec((2, _RB, IN_CH), lambda i: (0, i, 0)),
          pl.BlockSpec((_RB, 16), lambda i: (i, 0)),
          pl.BlockSpec((IN_CH, HID), lambda i: (0, 0)),
          pl.BlockSpec((1, HID), lambda i: (0, 0)),
          pl.BlockSpec((HID, OUT_CH), lambda i: (0, 0)),
      ],
      out_specs=pl.BlockSpec((_RB, OUT_CH), lambda i: (i, 0)),
      out_shape=jax.ShapeDtypeStruct((NPAD, OUT_CH), f32),
  )(u1, a, dinv16, W1, b1r, W2)

  c = _make_sc_prop(OUT_CH)(u2, src_p, dst_p, zeros64)

  out = pl.pallas_call(
      _tc_pool_body,
      grid=(grid,),
      in_specs=[
          pl.BlockSpec((_RB, OUT_CH), lambda i: (i, 0)),
          pl.BlockSpec((2, _RB, OUT_CH), lambda i: (0, i, 0)),
          pl.BlockSpec((_RB, 16), lambda i: (i, 0)),
          pl.BlockSpec((1, OUT_CH), lambda i: (0, 0)),
          pl.BlockSpec((_RB, 1), lambda i: (i, 0)),
      ],
      out_specs=pl.BlockSpec((G, OUT_CH), lambda i: (0, 0)),
      out_shape=jax.ShapeDtypeStruct((G, OUT_CH), f32),
      scratch_shapes=[pltpu.VMEM((G, 128), f32)],
  )(u2, c, dinv16, b2r, batch_p)
  return out
